# baseline (speedup 1.0000x reference)
import jax
import jax.numpy as jnp
from jax import lax
from jax.experimental import pallas as pl

_EPS = 1e-5
_CST = [2, 1, 1, 1, 1, 1, 1, 1, 1, 1]
_PST = [2, 1, 2, 1, 2, 1, 2, 1, 2, 1]


def _pool(x, ps):
    return lax.reduce_window(x, -jnp.inf, lax.max, (1, 2, 2, 1), (1, ps, ps, 1),
                             ((0, 0), (1, 1), (1, 1), (0, 0)))


def _conv_im2col(x, wk, cs):
    k = wk.shape[0]
    p = (k - 1) // 2
    xp = jnp.pad(x, ((0, 0), (p, p), (p, p), (0, 0)))
    B, Hp, Wp, C = xp.shape
    Ho = (x.shape[1] + 2 * p - k) // cs + 1
    Wo = (x.shape[2] + 2 * p - k) // cs + 1
    cols = []
    for dy in range(k):
        for dx in range(k):
            cols.append(lax.slice(xp, (0, dy, dx, 0),
                                  (B, dy + (Ho - 1) * cs + 1, dx + (Wo - 1) * cs + 1, C),
                                  (1, cs, cs, 1)))
    patches = jnp.concatenate(cols, axis=-1).reshape(B * Ho * Wo, k * k * C)
    wmat = wk.reshape(k * k * C, -1)
    out = jnp.dot(patches.astype(jnp.bfloat16), wmat.astype(jnp.bfloat16),
                  preferred_element_type=jnp.float32)
    return out.reshape(B, Ho, Wo, -1)


def _roi_pool_mask(feat, rois, out_h=8, out_w=16):
    B, C, Hf, Wf = feat.shape
    x0, y0 = rois[:, 0], rois[:, 1]
    x1 = jnp.maximum(jnp.minimum(rois[:, 2], Wf - 1), x0)
    y1 = jnp.maximum(jnp.minimum(rois[:, 3], Hf - 1), y0)
    H = y1 - y0 + 1
    W = x1 - x0 + 1

    def masks(lo, L, osize, isize):
        o = jnp.arange(osize)
        start = lo[:, None] + (o[None, :] * L[:, None]) // osize
        end = lo[:, None] + ((o[None, :] + 1) * L[:, None] + osize - 1) // osize
        idx = jnp.arange(isize)
        return (idx[None, None, :] >= start[:, :, None]) & (idx[None, None, :] < end[:, :, None])

    mh = masks(y0, H, out_h, Hf)
    mw = masks(x0, W, out_w, Wf)
    t = jnp.max(jnp.where(mh[:, None, :, :, None], feat[:, :, None, :, :], -jnp.inf), axis=3)
    return jnp.max(jnp.where(mw[:, None, None, :, :], t[:, :, :, None, :], -jnp.inf), axis=4)


def _rmat(s, h):
    return jnp.array([[s, 0, s, 0], [0, s, 0, s], [-h, 0, h, 0], [0, -h, 0, h]], jnp.float32)


def _pallas_touch(x):
    def _k(x_ref, o_ref):
        o_ref[...] = x_ref[...]
    return pl.pallas_call(_k, out_shape=jax.ShapeDtypeStruct(x.shape, x.dtype))(x)


def kernel(image, conv_ws, conv_bs, bn_g, bn_b, bn_mu, bn_var, cls_ws, cls_bs, head_ws, head_bs):
    x = jnp.transpose(image, (0, 2, 3, 1))
    feats = {}
    for i in range(10):
        inv = bn_g[i] / jnp.sqrt(bn_var[i] + _EPS)
        bias_eff = conv_bs[i] * inv + bn_b[i] - bn_mu[i] * inv
        weff = (conv_ws[i] * inv[:, None, None, None]).transpose(2, 3, 1, 0)
        z = _conv_im2col(x, weff, _CST[i]) + bias_eff[None, None, None, :]
        z = jax.nn.relu(z)
        x = _pool(z, _PST[i])
        if i in (1, 3, 5):
            feats[i] = jnp.transpose(x, (0, 3, 1, 2))
    h = jnp.transpose(x, (0, 3, 1, 2)).reshape(x.shape[0], -1)
    h = jax.nn.relu(h @ cls_ws[0].T + cls_bs[0])
    h = jax.nn.relu(h @ cls_ws[1].T + cls_bs[1])
    bb = h @ cls_ws[2].T + cls_bs[2]
    bb = _pallas_touch(bb)
    r1 = jnp.clip(bb @ _rmat(122.0, 61.0), 0.0, 122.0).astype(jnp.int32)
    r2 = jnp.clip(bb @ _rmat(63.0, 31.5), 0.0, 63.0).astype(jnp.int32)
    r3 = jnp.clip(bb @ _rmat(33.0, 16.5), 0.0, 33.0).astype(jnp.int32)
    p1 = _roi_pool_mask(feats[1], r1)
    p2 = _roi_pool_mask(feats[3], r2)
    p3 = _roi_pool_mask(feats[5], r3)
    roi = jnp.concatenate([p1, p2, p3], axis=1).reshape(bb.shape[0], -1)
    chars = tuple(roi @ w.T + b for w, b in zip(head_ws, head_bs))
    return (bb,) + chars


# speedup vs baseline: 8.0297x; 8.0297x over previous
import jax
import jax.numpy as jnp
from jax import lax
from jax.experimental import pallas as pl
from jax.experimental.pallas import tpu as pltpu

_EPS = 1e-5
_CST = [2, 1, 1, 1, 1, 1, 1, 1, 1, 1]
_PST = [2, 1, 2, 1, 2, 1, 2, 1, 2, 1]
_HEAD_DIMS = [38, 25, 35, 35, 35, 35, 35]


def _conv_block(x, W, b, g, beta, mu, var, cs, ps):
    k = W.shape[2]
    p = (k - 1) // 2
    x = lax.conv_general_dilated(x, W, (cs, cs), ((p, p), (p, p)),
                                 dimension_numbers=('NCHW', 'OIHW', 'NCHW'))
    x = x + b[None, :, None, None]
    inv = g / jnp.sqrt(var + _EPS)
    x = x * inv[None, :, None, None] + (beta - mu * inv)[None, :, None, None]
    x = jax.nn.relu(x)
    x = lax.reduce_window(x, -jnp.inf, lax.max, (1, 1, 2, 2), (1, 1, ps, ps),
                          ((0, 0), (0, 0), (1, 1), (1, 1)))
    return x


def _box_mat(s, h):
    return jnp.array([[s, 0, s, 0], [0, s, 0, s], [-h, 0, h, 0], [0, -h, 0, h]],
                     jnp.float32)


def _roi_pool_kernel(r_ref, f1_ref, f3_ref, f5_ref, o_ref):
    b = pl.program_id(0)
    neg = jnp.float32(-jnp.inf)

    def pool(f_ref, m, cbase):
        feat = f_ref[0]
        C, Hf, Wf = feat.shape
        x0 = r_ref[m, b, 0]
        y0 = r_ref[m, b, 1]
        x1 = jnp.maximum(jnp.minimum(r_ref[m, b, 2], Wf - 1), x0)
        y1 = jnp.maximum(jnp.minimum(r_ref[m, b, 3], Hf - 1), y0)
        Hl = y1 - y0 + 1
        Wl = x1 - x0 + 1
        ih = lax.broadcasted_iota(jnp.int32, (1, Hf, 1), 1)
        iw = lax.broadcasted_iota(jnp.int32, (16, Wf), 1)
        jj = lax.broadcasted_iota(jnp.int32, (16, Wf), 0)
        ws = x0 + (jj * Wl) // 16
        we = x0 + ((jj + 1) * Wl + 15) // 16
        mw = (iw >= ws) & (iw < we)
        for i in range(8):
            hs = y0 + (i * Hl) // 8
            he = y0 + ((i + 1) * Hl + 7) // 8
            mh = (ih >= hs) & (ih < he)
            ti = jnp.max(jnp.where(mh, feat, neg), axis=1)
            cell = jnp.where(mw[None, :, :], ti[:, None, :], neg)
            o_ref[0, cbase:cbase + C, i * 16:(i + 1) * 16] = jnp.max(cell, axis=2)

    pool(f1_ref, 0, 0)
    pool(f3_ref, 1, 64)
    pool(f5_ref, 2, 224)


def _roi_pool(f1, f3, f5, rois):
    B = f1.shape[0]
    return pl.pallas_call(
        _roi_pool_kernel,
        grid=(B,),
        in_specs=[
            pl.BlockSpec(memory_space=pltpu.SMEM),
            pl.BlockSpec((1,) + f1.shape[1:], lambda b: (b, 0, 0, 0)),
            pl.BlockSpec((1,) + f3.shape[1:], lambda b: (b, 0, 0, 0)),
            pl.BlockSpec((1,) + f5.shape[1:], lambda b: (b, 0, 0, 0)),
        ],
        out_specs=pl.BlockSpec((1, 416, 128), lambda b: (b, 0, 0)),
        out_shape=jax.ShapeDtypeStruct((B, 416, 128), jnp.float32),
        compiler_params=pltpu.CompilerParams(dimension_semantics=("parallel",)),
    )(rois, f1, f3, f5)


def _heads_kernel(w_ref, xt_ref, b_ref, o_ref):
    k = pl.program_id(1)
    part = lax.dot_general(w_ref[...], xt_ref[...], (((1,), (0,)), ((), ())),
                           preferred_element_type=jnp.float32)

    @pl.when(k == 0)
    def _():
        o_ref[...] = b_ref[...] + part

    @pl.when(k != 0)
    def _():
        o_ref[...] = o_ref[...] + part


def _heads(roi_t, Wcat, bcat):
    B = roi_t.shape[1]
    KT = 6656
    return pl.pallas_call(
        _heads_kernel,
        grid=(2, 8),
        in_specs=[
            pl.BlockSpec((120, KT), lambda h, k: (h, k)),
            pl.BlockSpec((KT, B), lambda h, k: (k, 0)),
            pl.BlockSpec((120, 1), lambda h, k: (h, 0)),
        ],
        out_specs=pl.BlockSpec((120, B), lambda h, k: (h, 0)),
        out_shape=jax.ShapeDtypeStruct((240, B), jnp.float32),
        compiler_params=pltpu.CompilerParams(
            dimension_semantics=("parallel", "arbitrary")),
    )(Wcat, roi_t, bcat)


def kernel(image, conv_ws, conv_bs, bn_g, bn_b, bn_mu, bn_var, cls_ws, cls_bs,
           head_ws, head_bs):
    feats = []
    x = image
    for i in range(10):
        x = _conv_block(x, conv_ws[i], conv_bs[i], bn_g[i], bn_b[i], bn_mu[i],
                        bn_var[i], _CST[i], _PST[i])
        feats.append(x)
    B = x.shape[0]
    h = x.reshape(B, -1)
    h = jax.nn.relu(h @ cls_ws[0].T + cls_bs[0])
    h = jax.nn.relu(h @ cls_ws[1].T + cls_bs[1])
    bb = h @ cls_ws[2].T + cls_bs[2]
    r1 = lax.stop_gradient(jnp.clip(bb @ _box_mat(122.0, 61.0), 0.0, 122.0)).astype(jnp.int32)
    r2 = lax.stop_gradient(jnp.clip(bb @ _box_mat(63.0, 31.5), 0.0, 63.0)).astype(jnp.int32)
    r3 = lax.stop_gradient(jnp.clip(bb @ _box_mat(33.0, 16.5), 0.0, 33.0)).astype(jnp.int32)
    rois = jnp.stack([r1, r2, r3])

    roi = _roi_pool(feats[1], feats[3], feats[5], rois)
    roi_t = roi.reshape(B, 53248).T

    pad = jnp.zeros((2, 53248), jnp.float32)
    Wcat = jnp.concatenate(head_ws + [pad], axis=0)
    bcat = jnp.concatenate(head_bs + [jnp.zeros((2,), jnp.float32)])[:, None]
    out_t = _heads(roi_t, Wcat, bcat)
    out = out_t[:238].T

    chars = []
    off = 0
    for d in _HEAD_DIMS:
        chars.append(out[:, off:off + d])
        off += d
    return (bb,) + tuple(chars)


# speedup vs baseline: 8.4723x; 1.0551x over previous
import jax
import jax.numpy as jnp
from jax import lax
from jax.experimental import pallas as pl
from jax.experimental.pallas import tpu as pltpu

_EPS = 1e-5
_CST = [2, 1, 1, 1, 1, 1, 1, 1, 1, 1]
_PST = [2, 1, 2, 1, 2, 1, 2, 1, 2, 1]
_HEAD_DIMS = [38, 25, 35, 35, 35, 35, 35]


def _conv_block(x, W, b, g, beta, mu, var, cs, ps):
    k = W.shape[2]
    p = (k - 1) // 2
    x = lax.conv_general_dilated(x, W, (cs, cs), ((p, p), (p, p)),
                                 dimension_numbers=('NCHW', 'OIHW', 'NCHW'))
    x = x + b[None, :, None, None]
    inv = g / jnp.sqrt(var + _EPS)
    x = x * inv[None, :, None, None] + (beta - mu * inv)[None, :, None, None]
    x = jax.nn.relu(x)
    x = lax.reduce_window(x, -jnp.inf, lax.max, (1, 1, 2, 2), (1, 1, ps, ps),
                          ((0, 0), (0, 0), (1, 1), (1, 1)))
    return x


def _box_mat(s, h):
    return jnp.array([[s, 0, s, 0], [0, s, 0, s], [-h, 0, h, 0], [0, -h, 0, h]],
                     jnp.float32)


def _roi_pool_kernel(r_ref, f1_ref, f3_ref, f5_ref, o_ref):
    b = pl.program_id(0)
    neg = jnp.float32(-jnp.inf)

    def pool(f_ref, m, cbase):
        feat = f_ref[0]
        C, Hf, Wf = feat.shape
        x0 = r_ref[m, b, 0]
        y0 = r_ref[m, b, 1]
        x1 = jnp.maximum(jnp.minimum(r_ref[m, b, 2], Wf - 1), x0)
        y1 = jnp.maximum(jnp.minimum(r_ref[m, b, 3], Hf - 1), y0)
        Hl = y1 - y0 + 1
        Wl = x1 - x0 + 1
        ih = lax.broadcasted_iota(jnp.int32, (1, Hf, 1), 1)
        iw = lax.broadcasted_iota(jnp.int32, (16, Wf), 1)
        jj = lax.broadcasted_iota(jnp.int32, (16, Wf), 0)
        ws = x0 + (jj * Wl) // 16
        we = x0 + ((jj + 1) * Wl + 15) // 16
        mw = (iw >= ws) & (iw < we)
        for i in range(8):
            hs = y0 + (i * Hl) // 8
            he = y0 + ((i + 1) * Hl + 7) // 8
            mh = (ih >= hs) & (ih < he)
            ti = jnp.max(jnp.where(mh, feat, neg), axis=1)
            cell = jnp.where(mw[None, :, :], ti[:, None, :], neg)
            o_ref[0, cbase:cbase + C, i * 16:(i + 1) * 16] = jnp.max(cell, axis=2)

    pool(f1_ref, 0, 0)
    pool(f3_ref, 1, 64)
    pool(f5_ref, 2, 224)


def _roi_pool(f1, f3, f5, rois):
    B = f1.shape[0]
    return pl.pallas_call(
        _roi_pool_kernel,
        grid=(B,),
        in_specs=[
            pl.BlockSpec(memory_space=pltpu.SMEM),
            pl.BlockSpec((1,) + f1.shape[1:], lambda b: (b, 0, 0, 0)),
            pl.BlockSpec((1,) + f3.shape[1:], lambda b: (b, 0, 0, 0)),
            pl.BlockSpec((1,) + f5.shape[1:], lambda b: (b, 0, 0, 0)),
        ],
        out_specs=pl.BlockSpec((1, 416, 128), lambda b: (b, 0, 0)),
        out_shape=jax.ShapeDtypeStruct((B, 416, 128), jnp.float32),
        compiler_params=pltpu.CompilerParams(dimension_semantics=("parallel",)),
    )(rois, f1, f3, f5)


def _heads_kernel(xt_ref, *refs):
    k = pl.program_id(0)
    x = xt_ref[...]
    for w_ref, b_ref, o_ref in zip(refs[:7], refs[7:14], refs[14:]):
        part = lax.dot_general(w_ref[...], x, (((1,), (0,)), ((), ())),
                               preferred_element_type=jnp.float32)

        @pl.when(k == 0)
        def _(part=part, b_ref=b_ref, o_ref=o_ref):
            o_ref[...] = b_ref[...] + part

        @pl.when(k != 0)
        def _(part=part, o_ref=o_ref):
            o_ref[...] = o_ref[...] + part


def _heads(roi_t, head_ws, head_bs):
    B = roi_t.shape[1]
    KT = 6656
    dims = [w.shape[0] for w in head_ws]
    return pl.pallas_call(
        _heads_kernel,
        grid=(8,),
        in_specs=(
            [pl.BlockSpec((KT, B), lambda k: (k, 0))]
            + [pl.BlockSpec((d, KT), lambda k: (0, k)) for d in dims]
            + [pl.BlockSpec((d, 1), lambda k: (0, 0)) for d in dims]
        ),
        out_specs=[pl.BlockSpec((d, B), lambda k: (0, 0)) for d in dims],
        out_shape=[jax.ShapeDtypeStruct((d, B), jnp.float32) for d in dims],
        compiler_params=pltpu.CompilerParams(
            dimension_semantics=("arbitrary",)),
    )(roi_t, *head_ws, *[b[:, None] for b in head_bs])


def kernel(image, conv_ws, conv_bs, bn_g, bn_b, bn_mu, bn_var, cls_ws, cls_bs,
           head_ws, head_bs):
    feats = []
    x = image
    for i in range(10):
        x = _conv_block(x, conv_ws[i], conv_bs[i], bn_g[i], bn_b[i], bn_mu[i],
                        bn_var[i], _CST[i], _PST[i])
        feats.append(x)
    B = x.shape[0]
    h = x.reshape(B, -1)
    h = jax.nn.relu(h @ cls_ws[0].T + cls_bs[0])
    h = jax.nn.relu(h @ cls_ws[1].T + cls_bs[1])
    bb = h @ cls_ws[2].T + cls_bs[2]
    r1 = lax.stop_gradient(jnp.clip(bb @ _box_mat(122.0, 61.0), 0.0, 122.0)).astype(jnp.int32)
    r2 = lax.stop_gradient(jnp.clip(bb @ _box_mat(63.0, 31.5), 0.0, 63.0)).astype(jnp.int32)
    r3 = lax.stop_gradient(jnp.clip(bb @ _box_mat(33.0, 16.5), 0.0, 33.0)).astype(jnp.int32)
    rois = jnp.stack([r1, r2, r3])

    roi = _roi_pool(feats[1], feats[3], feats[5], rois)
    roi_t = roi.reshape(B, 53248).T

    outs_t = _heads(roi_t, head_ws, head_bs)
    return (bb,) + tuple(o.T for o in outs_t)
